# baseline (speedup 1.0000x reference)
import functools

import jax
import jax.numpy as jnp
from jax import lax
from jax.experimental import pallas as pl
from jax.experimental.pallas import tpu as pltpu
from jax.experimental.pallas import tpu_sc as plsc

N = 65536
D = 32
K = 512
BZ = 2048

NSEG = 2
NS_ROWS = N // NSEG
NB = NS_ROWS // BZ

NC, NS = 2, 16
NW = NC * NS
BPW = NS_ROWS // NW
CHUNK = 128
NCH = BPW // CHUNK


def _vq_tc_body(z_ref, w_ref, idx_ref, loss_ref):
    i = pl.program_id(0)
    z = z_ref[...]
    w = w_ref[...]
    zn = jnp.sum(z * z, axis=1, keepdims=True)
    wn = jnp.sum(w * w, axis=1)
    mm = lax.dot_general(-2.0 * z, w, (((1,), (1,)), ((), ())),
                         preferred_element_type=jnp.float32)
    d = (zn + wn[None, :]) + mm
    mind = jnp.min(d, axis=1)
    ids = lax.broadcasted_iota(jnp.int32, d.shape, 1).astype(jnp.float32)
    idx_f = jnp.min(jnp.where(d == mind[:, None], ids, jnp.float32(K)), axis=1)
    idx = idx_f.astype(jnp.int32)
    idx_ref[...] = idx.reshape(BZ // BPW, NCH, CHUNK)

    @pl.when(i == 0)
    def _():
        loss_ref[0, 0] = 0.0

    total = loss_ref[0, 0] + jnp.sum(mind)
    loss_ref[0, 0] = total

    @pl.when(i == NB - 1)
    def _():
        loss_ref[0, 0] = total * (1.25 / (N * D))


def _tc_argmin(z_seg, W):
    return pl.pallas_call(
        _vq_tc_body,
        grid=(NB,),
        in_specs=[
            pl.BlockSpec((BZ, D), lambda i: (i, 0)),
            pl.BlockSpec((K, D), lambda i: (0, 0)),
        ],
        out_specs=[
            pl.BlockSpec((BZ // BPW, NCH, CHUNK), lambda i: (i, 0, 0)),
            pl.BlockSpec(block_shape=(1, 1), index_map=lambda i: (0, 0),
                         memory_space=pltpu.SMEM),
        ],
        out_shape=[
            jax.ShapeDtypeStruct((NW, NCH, CHUNK), jnp.int32),
            jax.ShapeDtypeStruct((1, 1), jnp.float32),
        ],
    )(z_seg, W)


def _sc_gather_body(table_hbm, idx_hbm, out_hbm, idxout_hbm, idx_v, rows_v,
                    sem):
    wid = lax.axis_index("s") * NC + lax.axis_index("c")
    pltpu.sync_copy(idx_hbm.at[wid], idx_v)
    copies = [
        pltpu.async_copy(table_hbm.at[idx_v.at[j]],
                         rows_v.at[pl.ds(j * CHUNK, CHUNK)], sem)
        for j in range(NCH)
    ]
    for j in range(NCH):
        pltpu.sync_copy(idx_v.at[j],
                        idxout_hbm.at[pl.ds(wid * BPW + j * CHUNK, CHUNK)])
    for c in copies:
        c.wait()
    pltpu.sync_copy(rows_v, out_hbm.at[pl.ds(wid * BPW, BPW)])


@functools.cache
def _sc_gather():
    return pl.kernel(
        _sc_gather_body,
        out_type=[jax.ShapeDtypeStruct((NS_ROWS, D), jnp.float32),
                  jax.ShapeDtypeStruct((NS_ROWS,), jnp.int32)],
        mesh=plsc.VectorSubcoreMesh(core_axis_name="c", subcore_axis_name="s"),
        scratch_types=[
            pltpu.VMEM((NCH, CHUNK), jnp.int32),
            pltpu.VMEM((BPW, D), jnp.float32),
            pltpu.SemaphoreType.DMA,
        ],
        compiler_params=pltpu.CompilerParams(use_tc_tiling_on_sc=False),
    )


def kernel(z, W):
    qs, ids, ls = [], [], []
    sc = None
    for s in range(NSEG):
        z_seg = lax.slice_in_dim(z, s * NS_ROWS, (s + 1) * NS_ROWS, axis=0)
        idx3, loss = _tc_argmin(z_seg, W)
        if sc is None:
            sc = _sc_gather()
        q, idx_flat = sc(W, idx3)
        qs.append(q)
        ids.append(idx_flat)
        ls.append(loss[0, 0])
    quantized = jnp.concatenate(qs, axis=0)
    idx = jnp.concatenate(ids, axis=0)
    return quantized, sum(ls), idx


assert BZ % BPW == 0 and NB * (BZ // BPW) == NW


# speedup vs baseline: 1.0709x; 1.0709x over previous
import functools

import jax
import jax.numpy as jnp
from jax import lax
from jax.experimental import pallas as pl
from jax.experimental.pallas import tpu as pltpu
from jax.experimental.pallas import tpu_sc as plsc

N = 65536
D = 32
K = 512
BZ = 2048

NSEG = 2
NS_ROWS = N // NSEG
NB = NS_ROWS // BZ

NC, NS = 2, 16
NW = NC * NS
BPW = NS_ROWS // NW
CHUNK = 128
NCH = BPW // CHUNK


def _vq_tc_body(z_ref, w_ref, idx_ref, loss_ref):
    i = pl.program_id(0)
    z = z_ref[...]
    w = w_ref[...]
    zn = jnp.sum(z * z, axis=1, keepdims=True)
    wn = jnp.sum(w * w, axis=1)
    mm = lax.dot_general(-2.0 * z, w, (((1,), (1,)), ((), ())),
                         preferred_element_type=jnp.float32)
    d = (zn + wn[None, :]) + mm
    mind = jnp.min(d, axis=1)
    ids = lax.broadcasted_iota(jnp.int32, d.shape, 1).astype(jnp.float32)
    idx_f = jnp.min(jnp.where(d == mind[:, None], ids, jnp.float32(K)), axis=1)
    idx = idx_f.astype(jnp.int32)
    idx_ref[...] = idx.reshape(BZ // BPW, NCH, CHUNK)

    @pl.when(i == 0)
    def _():
        loss_ref[0, 0] = 0.0

    total = loss_ref[0, 0] + jnp.sum(mind)
    loss_ref[0, 0] = total

    @pl.when(i == NB - 1)
    def _():
        loss_ref[0, 0] = total * (1.25 / (N * D))


def _tc_argmin(z, W, seg):
    return pl.pallas_call(
        _vq_tc_body,
        grid=(NB,),
        in_specs=[
            pl.BlockSpec((BZ, D), lambda i: (i + seg * NB, 0)),
            pl.BlockSpec((K, D), lambda i: (0, 0)),
        ],
        out_specs=[
            pl.BlockSpec((BZ // BPW, NCH, CHUNK), lambda i: (i, 0, 0)),
            pl.BlockSpec(block_shape=(1, 1), index_map=lambda i: (0, 0),
                         memory_space=pltpu.SMEM),
        ],
        out_shape=[
            jax.ShapeDtypeStruct((NW, NCH, CHUNK), jnp.int32),
            jax.ShapeDtypeStruct((1, 1), jnp.float32),
        ],
    )(z, W)


def _gather_seg(table_hbm, idx_v, rows_v, sem):
    copies = [
        pltpu.async_copy(table_hbm.at[idx_v.at[j]],
                         rows_v.at[pl.ds(j * CHUNK, CHUNK)], sem)
        for j in range(NCH)
    ]
    for c in copies:
        c.wait()


def _sc_gather0_body(table_hbm, idx_hbm, out_hbm, idxout_hbm, idx_v, rows_v,
                     sem):
    wid = lax.axis_index("s") * NC + lax.axis_index("c")
    pltpu.sync_copy(idx_hbm.at[wid], idx_v)
    _gather_seg(table_hbm, idx_v, rows_v, sem)
    pltpu.sync_copy(rows_v, out_hbm.at[pl.ds(wid * BPW, BPW)])
    for j in range(NCH):
        pltpu.sync_copy(idx_v.at[j],
                        idxout_hbm.at[pl.ds(wid * BPW + j * CHUNK, CHUNK)])


def _sc_gather1_body(table_hbm, idx_hbm, q0_hbm, i0_hbm, out_hbm, idxout_hbm,
                     idx_v, rows_v, sem):
    wid = lax.axis_index("s") * NC + lax.axis_index("c")
    base = wid * BPW
    pltpu.sync_copy(idx_hbm.at[wid], idx_v)
    _gather_seg(table_hbm, idx_v, rows_v, sem)
    pltpu.sync_copy(rows_v, out_hbm.at[pl.ds(NS_ROWS + base, BPW)])
    for j in range(NCH):
        pltpu.sync_copy(idx_v.at[j],
                        idxout_hbm.at[pl.ds(NS_ROWS + base + j * CHUNK, CHUNK)])
    pltpu.sync_copy(q0_hbm.at[pl.ds(base, BPW)], rows_v)
    pltpu.sync_copy(rows_v, out_hbm.at[pl.ds(base, BPW)])
    for j in range(NCH):
        pltpu.sync_copy(i0_hbm.at[pl.ds(base + j * CHUNK, CHUNK)],
                        idx_v.at[j])
        pltpu.sync_copy(idx_v.at[j],
                        idxout_hbm.at[pl.ds(base + j * CHUNK, CHUNK)])


@functools.cache
def _sc_gather0():
    return pl.kernel(
        _sc_gather0_body,
        out_type=[jax.ShapeDtypeStruct((NS_ROWS, D), jnp.float32),
                  jax.ShapeDtypeStruct((NS_ROWS,), jnp.int32)],
        mesh=plsc.VectorSubcoreMesh(core_axis_name="c", subcore_axis_name="s"),
        scratch_types=[
            pltpu.VMEM((NCH, CHUNK), jnp.int32),
            pltpu.VMEM((BPW, D), jnp.float32),
            pltpu.SemaphoreType.DMA,
        ],
        compiler_params=pltpu.CompilerParams(use_tc_tiling_on_sc=False),
    )


@functools.cache
def _sc_gather1():
    return pl.kernel(
        _sc_gather1_body,
        out_type=[jax.ShapeDtypeStruct((N, D), jnp.float32),
                  jax.ShapeDtypeStruct((N,), jnp.int32)],
        mesh=plsc.VectorSubcoreMesh(core_axis_name="c", subcore_axis_name="s"),
        scratch_types=[
            pltpu.VMEM((NCH, CHUNK), jnp.int32),
            pltpu.VMEM((BPW, D), jnp.float32),
            pltpu.SemaphoreType.DMA,
        ],
        compiler_params=pltpu.CompilerParams(use_tc_tiling_on_sc=False),
    )


def kernel(z, W):
    idx0, loss0 = _tc_argmin(z, W, 0)
    q0, i0 = _sc_gather0()(W, idx0)
    idx1, loss1 = _tc_argmin(z, W, 1)
    quantized, idx = _sc_gather1()(W, idx1, q0, i0)
    return quantized, loss0[0, 0] + loss1[0, 0], idx


assert BZ % BPW == 0 and NB * (BZ // BPW) == NW


# speedup vs baseline: 1.0848x; 1.0130x over previous
import functools

import jax
import jax.numpy as jnp
from jax import lax
from jax.experimental import pallas as pl
from jax.experimental.pallas import tpu as pltpu
from jax.experimental.pallas import tpu_sc as plsc

N = 65536
D = 32
K = 512
BZ = 2048
NB = N // BZ

NC, NS = 2, 16
NW = NC * NS
BPW = N // NW
CHUNK = 128
NCH = BPW // CHUNK


def _vq_tc_body(z_ref, w_ref, idx_ref, loss_ref):
    i = pl.program_id(0)
    z = z_ref[...]
    w = w_ref[...]
    zn = jnp.sum(z * z, axis=1, keepdims=True)
    wn = jnp.sum(w * w, axis=1)
    mm = lax.dot_general(-2.0 * z, w, (((1,), (1,)), ((), ())),
                         preferred_element_type=jnp.float32)
    d = (zn + wn[None, :]) + mm
    mind = jnp.min(d, axis=1)
    ids = lax.broadcasted_iota(jnp.int32, d.shape, 1).astype(jnp.float32)
    idx_f = jnp.min(jnp.where(d == mind[:, None], ids, jnp.float32(K)), axis=1)
    idx = idx_f.astype(jnp.int32)
    idx_ref[0, :, :] = idx.reshape(BZ // CHUNK, CHUNK)

    @pl.when(i == 0)
    def _():
        loss_ref[0, 0] = 0.0

    total = loss_ref[0, 0] + jnp.sum(mind)
    loss_ref[0, 0] = total

    @pl.when(i == NB - 1)
    def _():
        loss_ref[0, 0] = total * (1.25 / (N * D))


def _tc_argmin(z, W):
    return pl.pallas_call(
        _vq_tc_body,
        grid=(NB,),
        in_specs=[
            pl.BlockSpec((BZ, D), lambda i: (i, 0)),
            pl.BlockSpec((K, D), lambda i: (0, 0)),
        ],
        out_specs=[
            pl.BlockSpec((1, BZ // CHUNK, CHUNK), lambda i: (i, 0, 0)),
            pl.BlockSpec(block_shape=(1, 1), index_map=lambda i: (0, 0),
                         memory_space=pltpu.SMEM),
        ],
        out_shape=[
            jax.ShapeDtypeStruct((NB, BZ // CHUNK, CHUNK), jnp.int32),
            jax.ShapeDtypeStruct((1, 1), jnp.float32),
        ],
    )(z, W)


def _sc_gather_body(table_hbm, idx_hbm, out_hbm, idxout_hbm, idx_v, rows_v,
                    sem):
    wid = lax.axis_index("s") * NC + lax.axis_index("c")
    pltpu.sync_copy(idx_hbm.at[wid], idx_v)
    copies = [
        pltpu.async_copy(table_hbm.at[idx_v.at[j]],
                         rows_v.at[pl.ds(j * CHUNK, CHUNK)], sem)
        for j in range(NCH)
    ]
    for j in range(NCH):
        pltpu.sync_copy(idx_v.at[j],
                        idxout_hbm.at[pl.ds(wid * BPW + j * CHUNK, CHUNK)])
    for c in copies:
        c.wait()
    pltpu.sync_copy(rows_v, out_hbm.at[pl.ds(wid * BPW, BPW)])


@functools.cache
def _sc_gather():
    return pl.kernel(
        _sc_gather_body,
        out_type=[jax.ShapeDtypeStruct((N, D), jnp.float32),
                  jax.ShapeDtypeStruct((N,), jnp.int32)],
        mesh=plsc.VectorSubcoreMesh(core_axis_name="c", subcore_axis_name="s"),
        scratch_types=[
            pltpu.VMEM((NCH, CHUNK), jnp.int32),
            pltpu.VMEM((BPW, D), jnp.float32),
            pltpu.SemaphoreType.DMA,
        ],
        compiler_params=pltpu.CompilerParams(use_tc_tiling_on_sc=False),
    )


def kernel(z, W):
    idx3, loss = _tc_argmin(z, W)
    quantized, idx_flat = _sc_gather()(W, idx3)
    return quantized, loss[0, 0], idx_flat


assert (NB, BZ // CHUNK, CHUNK) == (NW, NCH, CHUNK)
